# Initial kernel scaffold; baseline (speedup 1.0000x reference)
import jax
import jax.numpy as jnp
from jax.experimental import pallas as pl


def kernel(boxes_offset, image_sizes):
    raise NotImplementedError("write your pallas kernel here")


# speedup vs baseline: 1.8557x; 1.8557x over previous
import functools

import jax
import jax.numpy as jnp
import numpy as np
from jax import lax
from jax.experimental import pallas as pl
from jax.experimental.pallas import tpu as pltpu

_ANCHORS = ((1.19, 1.98), (2.79, 4.59), (4.53, 8.92), (8.06, 5.29),
            (10.32, 10.65))
_A = 5
_STRIDE = 85
_NC = 80
_HW = 1024
_W = 32
_K = 300
_S = 512
_NMS_THRESH = 0.45
_DS = 32.0

_HIGH = jax.lax.Precision.HIGHEST


def _dotg(a, b):
    return lax.dot_general(a, b, ((( 1,), (1,)), ((), ())),
                           precision=_HIGH, preferred_element_type=jnp.float32)


def _yolo_kernel(x_ref, sf_ref, sz_ref, tri_ref, out_ref):
    h_img = sz_ref[0, 0, 0]
    w_img = sz_ref[0, 0, 1]

    sf = sf_ref[0]
    lanef = lax.broadcasted_iota(jnp.int32, (_A, _HW), 1).astype(jnp.float32)
    rowf = lax.broadcasted_iota(jnp.int32, (_A, _HW), 0).astype(jnp.float32)
    idxf = lanef * 5.0 + rowf

    def bs_body(_, carry):
        lo, hi = carry
        mid = (lo + hi) * 0.5
        cnt = jnp.sum((sf >= mid).astype(jnp.float32))
        ok = cnt >= 300.0
        return jnp.where(ok, mid, lo), jnp.where(ok, hi, mid)

    v, _ = lax.fori_loop(0, 40, bs_body,
                         (jnp.float32(0.0), jnp.float32(1.5)))

    n_gt = jnp.sum((sf > v).astype(jnp.float32))
    need = 300.0 - n_gt
    tied = sf == v

    def bs2_body(_, carry):
        lo, hi = carry
        mid = jnp.floor((lo + hi) * 0.5)
        cnt = jnp.sum((tied & (idxf <= mid)).astype(jnp.float32))
        ok = cnt >= need
        return jnp.where(ok, lo, mid), jnp.where(ok, mid, hi)

    _, cut = lax.fori_loop(0, 14, bs2_body,
                           (jnp.float32(-1.0), jnp.float32(5119.0)))

    sel = (sf > v) | (tied & (idxf <= cut))
    self_f = sel.astype(jnp.float32)

    prefix = jnp.dot(self_f, tri_ref[...], precision=_HIGH,
                     preferred_element_type=jnp.float32)
    rowtot = jnp.sum(self_f, axis=1, keepdims=True)
    s5r = lax.broadcasted_iota(jnp.int32, (_A, _A), 0)
    s5c = lax.broadcasted_iota(jnp.int32, (_A, _A), 1)
    s5 = (s5c < s5r).astype(jnp.float32)
    offs = jnp.dot(s5, rowtot, precision=_HIGH,
                   preferred_element_type=jnp.float32)
    prefix = prefix + offs

    lane1 = lax.broadcasted_iota(jnp.int32, (_HW,), 0)
    fx = jnp.bitwise_and(lane1, 31).astype(jnp.float32)
    fy = lax.shift_right_logical(lane1, 5).astype(jnp.float32)

    x1s, y1s, x2s, y2s, lbls = [], [], [], [], []
    for a in range(_A):
        base = a * _STRIDE
        logits = x_ref[0, base:base + _NC, :]
        m = jnp.max(logits, axis=0)
        r80 = lax.broadcasted_iota(jnp.int32, (_NC, _HW), 0).astype(jnp.float32)
        lbl = jnp.min(jnp.where(logits == m[None, :], r80, 1e9), axis=0)
        lbls.append(lbl)

        zx = x_ref[0, base + _NC, :]
        zy = x_ref[0, base + _NC + 1, :]
        zw = x_ref[0, base + _NC + 2, :]
        zh = x_ref[0, base + _NC + 3, :]
        sigx = 1.0 / (1.0 + jnp.exp(-zx))
        sigy = 1.0 / (1.0 + jnp.exp(-zy))
        cx = (fx + sigx) * _DS
        cy = (fy + sigy) * _DS
        aw, ah = _ANCHORS[a]
        bw = aw * jnp.exp(zw) * _DS
        bh = ah * jnp.exp(zh) * _DS
        x1s.append(jnp.clip(cx - bw / 2.0, 0.0, w_img))
        x2s.append(jnp.clip(cx + bw / 2.0, 0.0, w_img))
        y1s.append(jnp.clip(cy - bh / 2.0, 0.0, h_img))
        y2s.append(jnp.clip(cy + bh / 2.0, 0.0, h_img))

    riota = lax.broadcasted_iota(jnp.int32, (_S, _S), 0).astype(jnp.float32)
    ciota = lax.broadcasted_iota(jnp.int32, (_S, _S), 1).astype(jnp.float32)
    acc = jnp.zeros((8, _S), dtype=jnp.float32)
    for k in range(2 * _A):
        a, h = k // 2, k % 2
        sl = slice(h * _S, (h + 1) * _S)
        pc = prefix[a, sl][None, :]
        sc = self_f[a, sl][None, :]
        oh = jnp.where(pc == riota, sc, 0.0)
        p8 = jnp.concatenate([
            x1s[a][sl][None, :], y1s[a][sl][None, :],
            x2s[a][sl][None, :], y2s[a][sl][None, :],
            sf[a, sl][None, :], lbls[a][sl][None, :],
            idxf[a, sl][None, :] + 1.0,
            jnp.zeros((1, _S), jnp.float32),
        ], axis=0)
        acc = acc + _dotg(p8, oh)

    ident = jnp.where(riota == ciota, 1.0, 0.0)
    score_r = acc[4][None, :]
    idxp1_r = acc[6][None, :]
    idx_eff = jnp.where(idxp1_r > 0.0, idxp1_r, 8192.0)
    score_c = _dotg(ident, score_r)
    idx_c = _dotg(ident, idx_eff)
    cmp = (score_c > score_r) | ((score_c == score_r) & (idx_c < idx_eff))
    rank = jnp.sum(cmp.astype(jnp.float32), axis=0, keepdims=True)
    roh = jnp.where(rank == riota, 1.0, 0.0)
    srt = _dotg(acc, roh)

    lb = srt[5][None, :]
    off = lb * 4096.0
    bx1 = srt[0][None, :] + off
    by1 = srt[1][None, :] + off
    bx2 = srt[2][None, :] + off
    by2 = srt[3][None, :] + off
    bx1c = _dotg(ident, bx1)
    by1c = _dotg(ident, by1)
    bx2c = _dotg(ident, bx2)
    by2c = _dotg(ident, by2)
    area_r = jnp.maximum(bx2 - bx1, 0.0) * jnp.maximum(by2 - by1, 0.0)
    area_c = jnp.maximum(bx2c - bx1c, 0.0) * jnp.maximum(by2c - by1c, 0.0)
    ix1 = jnp.maximum(bx1c, bx1)
    iy1 = jnp.maximum(by1c, by1)
    ix2 = jnp.minimum(bx2c, bx2)
    iy2 = jnp.minimum(by2c, by2)
    inter = jnp.maximum(ix2 - ix1, 0.0) * jnp.maximum(iy2 - iy1, 0.0)
    iou = inter / (area_c + area_r - inter + 1e-9)
    iou = jnp.where(riota < ciota, iou, 0.0)
    supp = jnp.max(iou, axis=0, keepdims=True)

    scv = srt[4][None, :]
    keep = (supp <= _NMS_THRESH) & (scv > 0.0)
    keepf = keep.astype(jnp.float32)
    out_ref[0] = jnp.concatenate([
        srt[0][None, :] * keepf, srt[1][None, :] * keepf,
        srt[2][None, :] * keepf, srt[3][None, :] * keepf,
        scv * keepf,
        jnp.where(keep, lb, -1.0),
        srt[6][None, :],
        jnp.zeros((1, _S), jnp.float32),
    ], axis=0)


_TRI = np.triu(np.ones((_HW, _HW), dtype=np.float32), k=1)


@jax.jit
def kernel(boxes_offset, image_sizes):
    n, c, hh, ww = boxes_offset.shape
    x = boxes_offset.reshape(n, c, hh * ww)
    obj = x[:, _NC + 4::_STRIDE, :]
    sc = jax.nn.sigmoid(obj)
    sf = jnp.where(sc > 0.5, sc, 0.0)
    sz = image_sizes.astype(jnp.float32).reshape(n, 1, 2)

    out = pl.pallas_call(
        _yolo_kernel,
        grid=(n,),
        in_specs=[
            pl.BlockSpec((1, c, hh * ww), lambda i: (i, 0, 0)),
            pl.BlockSpec((1, _A, hh * ww), lambda i: (i, 0, 0)),
            pl.BlockSpec((1, 1, 2), lambda i: (i, 0, 0),
                         memory_space=pltpu.SMEM),
            pl.BlockSpec((_HW, _HW), lambda i: (0, 0)),
        ],
        out_specs=pl.BlockSpec((1, 8, _S), lambda i: (i, 0, 0)),
        out_shape=jax.ShapeDtypeStruct((n, 8, _S), jnp.float32),
        compiler_params=pltpu.CompilerParams(
            dimension_semantics=("arbitrary",)),
    )(x, sf, sz, jnp.asarray(_TRI))

    boxes = jnp.transpose(out[:, 0:4, :_K], (0, 2, 1))
    scores = out[:, 4, :_K]
    labels = out[:, 5, :_K].astype(jnp.int32)
    return boxes, scores, labels


# speedup vs baseline: 2.6627x; 1.4349x over previous
import functools

import jax
import jax.numpy as jnp
import numpy as np
from jax import lax
from jax.experimental import pallas as pl
from jax.experimental.pallas import tpu as pltpu

_ANCHORS = ((1.19, 1.98), (2.79, 4.59), (4.53, 8.92), (8.06, 5.29),
            (10.32, 10.65))
_A = 5
_STRIDE = 85
_NC = 80
_HW = 1024
_W = 32
_K = 300
_S = 384
_HALF = 512
_NMS_THRESH = 0.45
_DS = 32.0

_HIGH = jax.lax.Precision.HIGHEST


def _dotg(a, b):
    b1 = a.astype(jnp.bfloat16)
    r1 = a - b1.astype(jnp.float32)
    b2 = r1.astype(jnp.bfloat16)
    b3 = (r1 - b2.astype(jnp.float32)).astype(jnp.bfloat16)
    bb = b.astype(jnp.bfloat16)

    def one(p):
        return lax.dot_general(p, bb, (((1,), (1,)), ((), ())),
                               preferred_element_type=jnp.float32)

    return one(b1) + one(b2) + one(b3)


def _dotg_t(h, p):
    b1 = p.astype(jnp.bfloat16)
    r1 = p - b1.astype(jnp.float32)
    b2 = r1.astype(jnp.bfloat16)
    b3 = (r1 - b2.astype(jnp.float32)).astype(jnp.bfloat16)
    hb = h.astype(jnp.bfloat16)

    def one(q):
        return lax.dot_general(hb, q, (((1,), (1,)), ((), ())),
                               preferred_element_type=jnp.float32)

    return one(b1) + one(b2) + one(b3)


def _yolo_kernel(x_ref, sf_ref, sz_ref, out_ref):
    h_img = sz_ref[0, 0, 0]
    w_img = sz_ref[0, 0, 1]

    sf = sf_ref[0]
    lanef = lax.broadcasted_iota(jnp.int32, (_A, _HW), 1).astype(jnp.float32)
    rowf = lax.broadcasted_iota(jnp.int32, (_A, _HW), 0).astype(jnp.float32)
    idxf = lanef * 5.0 + rowf

    def bs_body(_, carry):
        lo, hi = carry
        mid = (lo + hi) * 0.5
        cnt = jnp.sum((sf >= mid).astype(jnp.float32))
        ok = cnt >= 300.0
        return jnp.where(ok, mid, lo), jnp.where(ok, hi, mid)

    v, _ = lax.fori_loop(0, 30, bs_body,
                         (jnp.float32(0.0), jnp.float32(1.5)))

    n_gt = jnp.sum((sf > v).astype(jnp.float32))
    need = 300.0 - n_gt
    tied = sf == v

    def bs2_body(_, carry):
        lo, hi = carry
        mid = jnp.floor((lo + hi) * 0.5)
        cnt = jnp.sum((tied & (idxf <= mid)).astype(jnp.float32))
        ok = cnt >= need
        return jnp.where(ok, lo, mid), jnp.where(ok, mid, hi)

    _, cut = lax.fori_loop(0, 14, bs2_body,
                           (jnp.float32(-1.0), jnp.float32(5119.0)))

    sel = (sf > v) | (tied & (idxf <= cut))
    self_f = sel.astype(jnp.float32)

    lanei5 = lax.broadcasted_iota(jnp.int32, (_A, _HW), 1)
    run = self_f
    for d in (1, 2, 4, 8, 16, 32, 64, 128, 256, 512):
        run = run + jnp.where(lanei5 >= d, pltpu.roll(run, d, 1), 0.0)
    prefix = run - self_f
    rowtot = run[:, _HW - 1:_HW]
    s5r = lax.broadcasted_iota(jnp.int32, (_A, _A), 0)
    s5c = lax.broadcasted_iota(jnp.int32, (_A, _A), 1)
    s5 = (s5c < s5r).astype(jnp.float32)
    offs = jnp.dot(s5, rowtot, precision=_HIGH,
                   preferred_element_type=jnp.float32)
    prefix = prefix + offs

    lane1 = lax.broadcasted_iota(jnp.int32, (_HW,), 0)
    fx = jnp.bitwise_and(lane1, 31).astype(jnp.float32)
    fy = lax.shift_right_logical(lane1, 5).astype(jnp.float32)

    x1s, y1s, x2s, y2s, lbls = [], [], [], [], []
    for a in range(_A):
        base = a * _STRIDE
        logits = x_ref[0, base:base + _NC, :]
        m = jnp.max(logits, axis=0)
        r80 = lax.broadcasted_iota(jnp.int32, (_NC, _HW), 0).astype(jnp.float32)
        lbl = jnp.min(jnp.where(logits == m[None, :], r80, 1e9), axis=0)
        lbls.append(lbl)

        zx = x_ref[0, base + _NC, :]
        zy = x_ref[0, base + _NC + 1, :]
        zw = x_ref[0, base + _NC + 2, :]
        zh = x_ref[0, base + _NC + 3, :]
        sigx = 1.0 / (1.0 + jnp.exp(-zx))
        sigy = 1.0 / (1.0 + jnp.exp(-zy))
        cx = (fx + sigx) * _DS
        cy = (fy + sigy) * _DS
        aw, ah = _ANCHORS[a]
        bw = aw * jnp.exp(zw) * _DS
        bh = ah * jnp.exp(zh) * _DS
        x1s.append(jnp.clip(cx - bw / 2.0, 0.0, w_img))
        x2s.append(jnp.clip(cx + bw / 2.0, 0.0, w_img))
        y1s.append(jnp.clip(cy - bh / 2.0, 0.0, h_img))
        y2s.append(jnp.clip(cy + bh / 2.0, 0.0, h_img))

    riota = lax.broadcasted_iota(jnp.int32, (_S, _S), 0).astype(jnp.float32)
    ciota = lax.broadcasted_iota(jnp.int32, (_S, _S), 1).astype(jnp.float32)
    riota_c = lax.broadcasted_iota(jnp.int32, (_S, _HALF), 0).astype(jnp.float32)
    acc = jnp.zeros((8, _S), dtype=jnp.float32)
    for k in range(2 * _A):
        a, h = k // 2, k % 2
        sl = slice(h * _HALF, (h + 1) * _HALF)
        pc = prefix[a, sl][None, :]
        sc = self_f[a, sl][None, :]
        oh = jnp.where(pc == riota_c, sc, 0.0)
        p8 = jnp.concatenate([
            x1s[a][sl][None, :], y1s[a][sl][None, :],
            x2s[a][sl][None, :], y2s[a][sl][None, :],
            sf[a, sl][None, :], lbls[a][sl][None, :],
            idxf[a, sl][None, :] + 1.0,
            jnp.zeros((1, _HALF), jnp.float32),
        ], axis=0)
        acc = acc + _dotg(p8, oh)

    ident = jnp.where(riota == ciota, 1.0, 0.0)
    score_r = acc[4][None, :]
    idxp1_r = acc[6][None, :]
    idx_eff = jnp.where(idxp1_r > 0.0, idxp1_r, 8192.0)
    colsA = _dotg_t(ident, jnp.concatenate([score_r, idx_eff], axis=0))
    score_c = colsA[:, 0:1]
    idx_c = colsA[:, 1:2]
    cmp = (score_c > score_r) | ((score_c == score_r) & (idx_c < idx_eff))
    rank = jnp.sum(cmp.astype(jnp.float32), axis=0, keepdims=True)
    roh = jnp.where(rank == riota, 1.0, 0.0)
    srt = _dotg(acc, roh)

    lb = srt[5][None, :]
    off = lb * 4096.0
    bx1 = srt[0][None, :] + off
    by1 = srt[1][None, :] + off
    bx2 = srt[2][None, :] + off
    by2 = srt[3][None, :] + off
    colsB = _dotg_t(ident, jnp.concatenate([bx1, by1, bx2, by2], axis=0))
    bx1c = colsB[:, 0:1]
    by1c = colsB[:, 1:2]
    bx2c = colsB[:, 2:3]
    by2c = colsB[:, 3:4]
    area_r = jnp.maximum(bx2 - bx1, 0.0) * jnp.maximum(by2 - by1, 0.0)
    area_c = jnp.maximum(bx2c - bx1c, 0.0) * jnp.maximum(by2c - by1c, 0.0)
    ix1 = jnp.maximum(bx1c, bx1)
    iy1 = jnp.maximum(by1c, by1)
    ix2 = jnp.minimum(bx2c, bx2)
    iy2 = jnp.minimum(by2c, by2)
    inter = jnp.maximum(ix2 - ix1, 0.0) * jnp.maximum(iy2 - iy1, 0.0)
    iou = inter / (area_c + area_r - inter + 1e-9)
    iou = jnp.where(riota < ciota, iou, 0.0)
    supp = jnp.max(iou, axis=0, keepdims=True)

    scv = srt[4][None, :]
    keep = (supp <= _NMS_THRESH) & (scv > 0.0)
    keepf = keep.astype(jnp.float32)
    out_ref[0] = jnp.concatenate([
        srt[0][None, :] * keepf, srt[1][None, :] * keepf,
        srt[2][None, :] * keepf, srt[3][None, :] * keepf,
        scv * keepf,
        jnp.where(keep, lb, -1.0),
        srt[6][None, :],
        jnp.zeros((1, _S), jnp.float32),
    ], axis=0)


@jax.jit
def kernel(boxes_offset, image_sizes):
    n, c, hh, ww = boxes_offset.shape
    x = boxes_offset.reshape(n, c, hh * ww)
    obj = x[:, _NC + 4::_STRIDE, :]
    sc = jax.nn.sigmoid(obj)
    sf = jnp.where(sc > 0.5, sc, 0.0)
    sz = image_sizes.astype(jnp.float32).reshape(n, 1, 2)

    out = pl.pallas_call(
        _yolo_kernel,
        grid=(n,),
        in_specs=[
            pl.BlockSpec((1, c, hh * ww), lambda i: (i, 0, 0)),
            pl.BlockSpec((1, _A, hh * ww), lambda i: (i, 0, 0)),
            pl.BlockSpec((1, 1, 2), lambda i: (i, 0, 0),
                         memory_space=pltpu.SMEM),
        ],
        out_specs=pl.BlockSpec((1, 8, _S), lambda i: (i, 0, 0)),
        out_shape=jax.ShapeDtypeStruct((n, 8, _S), jnp.float32),
        compiler_params=pltpu.CompilerParams(
            dimension_semantics=("arbitrary",)),
    )(x, sf, sz)

    boxes = jnp.transpose(out[:, 0:4, :_K], (0, 2, 1))
    scores = out[:, 4, :_K]
    labels = out[:, 5, :_K].astype(jnp.int32)
    return boxes, scores, labels


# speedup vs baseline: 3.9599x; 1.4872x over previous
import jax
import jax.numpy as jnp
import numpy as np
from jax import lax
from jax.experimental import pallas as pl
from jax.experimental.pallas import tpu as pltpu

_ANCHORS = ((1.19, 1.98), (2.79, 4.59), (4.53, 8.92), (8.06, 5.29),
            (10.32, 10.65))
_A = 5
_STRIDE = 85
_NC = 80
_HW = 1024
_K = 300
_S = 384
_NMS_THRESH = 0.45
_DS = 32.0


def _split3(a):
    b1 = a.astype(jnp.bfloat16)
    r1 = a - b1.astype(jnp.float32)
    b2 = r1.astype(jnp.bfloat16)
    b3 = (r1 - b2.astype(jnp.float32)).astype(jnp.bfloat16)
    return jnp.concatenate([b1, b2, b3], axis=0)


def _dotp(a, b):
    m = a.shape[0]
    o = lax.dot_general(_split3(a), b.astype(jnp.bfloat16),
                        (((1,), (1,)), ((), ())),
                        preferred_element_type=jnp.float32)
    return o[:m] + o[m:2 * m] + o[2 * m:3 * m]


def _dott(h, p):
    n = p.shape[0]
    o = lax.dot_general(h.astype(jnp.bfloat16), _split3(p),
                        (((1,), (1,)), ((), ())),
                        preferred_element_type=jnp.float32)
    return o[:, :n] + o[:, n:2 * n] + o[:, 2 * n:3 * n]


def _lane_prefix_excl(x, width):
    lanei = lax.broadcasted_iota(jnp.int32, x.shape, 1)
    run = x
    d = 1
    while d < width:
        run = run + jnp.where(lanei >= d, pltpu.roll(run, d, 1), 0.0)
        d *= 2
    return run - x, run


def _row_prefix_excl(x, height):
    rowi = lax.broadcasted_iota(jnp.int32, x.shape, 0)
    run = x
    d = 1
    while d < height:
        run = run + jnp.where(rowi >= d, pltpu.roll(run, d, 0), 0.0)
        d *= 2
    return run - x


def _yolo_kernel(x_ref, sf_ref, sz_ref, out_ref):
    h_img = sz_ref[0, 0, 0]
    w_img = sz_ref[0, 0, 1]

    sf = sf_ref[0]
    lanef = lax.broadcasted_iota(jnp.int32, (_A, _HW), 1).astype(jnp.float32)
    rowf = lax.broadcasted_iota(jnp.int32, (_A, _HW), 0).astype(jnp.float32)
    idxf = lanef * 5.0 + rowf

    def bs_body(_, carry):
        lo, hi = carry
        mid = (lo + hi) * 0.5
        cnt = jnp.sum((sf >= mid).astype(jnp.float32))
        ok = cnt >= 300.0
        return jnp.where(ok, mid, lo), jnp.where(ok, hi, mid)

    v, _ = lax.fori_loop(0, 27, bs_body,
                         (jnp.float32(0.0), jnp.float32(1.0)))

    n_gt = jnp.sum((sf > v).astype(jnp.float32))
    need = 300.0 - n_gt
    tied = sf == v
    tied_f = tied.astype(jnp.float32)

    colsum = jnp.sum(tied_f, axis=0, keepdims=True)
    colpre, _ = _lane_prefix_excl(colsum, _HW)
    rowpre = _row_prefix_excl(tied_f, _A)
    tie_rank = colpre + rowpre
    sel = (sf > v) | (tied & (tie_rank < need))
    self_f = sel.astype(jnp.float32)

    prefix, run = _lane_prefix_excl(self_f, _HW)
    rowtot = run[:, _HW - 1:_HW]
    offs = _row_prefix_excl(rowtot, _A)
    prefix = prefix + offs

    lane1 = lax.broadcasted_iota(jnp.int32, (_HW,), 0)
    fx = jnp.bitwise_and(lane1, 31).astype(jnp.float32)
    fy = lax.shift_right_logical(lane1, 5).astype(jnp.float32)

    x1s, y1s, x2s, y2s, lbls = [], [], [], [], []
    for a in range(_A):
        base = a * _STRIDE
        logits = x_ref[0, base:base + _NC, :]
        m = jnp.max(logits, axis=0)
        r80 = lax.broadcasted_iota(jnp.int32, (_NC, _HW), 0).astype(jnp.float32)
        lbl = jnp.min(jnp.where(logits == m[None, :], r80, 1e9), axis=0)
        lbls.append(lbl)

        zx = x_ref[0, base + _NC, :]
        zy = x_ref[0, base + _NC + 1, :]
        zw = x_ref[0, base + _NC + 2, :]
        zh = x_ref[0, base + _NC + 3, :]
        sigx = 1.0 / (1.0 + jnp.exp(-zx))
        sigy = 1.0 / (1.0 + jnp.exp(-zy))
        cx = (fx + sigx) * _DS
        cy = (fy + sigy) * _DS
        aw, ah = _ANCHORS[a]
        bw = aw * jnp.exp(zw) * _DS
        bh = ah * jnp.exp(zh) * _DS
        x1s.append(jnp.clip(cx - bw / 2.0, 0.0, w_img))
        x2s.append(jnp.clip(cx + bw / 2.0, 0.0, w_img))
        y1s.append(jnp.clip(cy - bh / 2.0, 0.0, h_img))
        y2s.append(jnp.clip(cy + bh / 2.0, 0.0, h_img))

    riota = lax.broadcasted_iota(jnp.int32, (_S, _S), 0).astype(jnp.float32)
    ciota = lax.broadcasted_iota(jnp.int32, (_S, _S), 1).astype(jnp.float32)
    riota_k = lax.broadcasted_iota(jnp.int32, (_S, _HW), 0).astype(jnp.float32)
    acc = jnp.zeros((8, _S), dtype=jnp.float32)
    for a in range(_A):
        pc = prefix[a][None, :]
        sc = self_f[a][None, :]
        oh = jnp.where(pc == riota_k, sc, 0.0)
        p8 = jnp.concatenate([
            x1s[a][None, :], y1s[a][None, :],
            x2s[a][None, :], y2s[a][None, :],
            sf[a][None, :], lbls[a][None, :],
            idxf[a][None, :] + 1.0,
            jnp.zeros((1, _HW), jnp.float32),
        ], axis=0)
        acc = acc + _dotp(p8, oh)

    ident = jnp.where(riota == ciota, 1.0, 0.0)
    score_r = acc[4][None, :]
    idxp1_r = acc[6][None, :]
    idx_eff = jnp.where(idxp1_r > 0.0, idxp1_r, 8192.0)
    colsA = _dott(ident, jnp.concatenate([score_r, idx_eff], axis=0))
    score_c = colsA[:, 0:1]
    idx_c = colsA[:, 1:2]
    above = (score_c > score_r) | ((score_c == score_r) & (idx_c < idx_eff))
    rank = jnp.sum(above.astype(jnp.float32), axis=0, keepdims=True)

    lb = acc[5][None, :]
    off = lb * 4096.0
    bx1 = acc[0][None, :] + off
    by1 = acc[1][None, :] + off
    bx2 = acc[2][None, :] + off
    by2 = acc[3][None, :] + off
    colsB = _dott(ident, jnp.concatenate([bx1, by1, bx2, by2], axis=0))
    bx1c = colsB[:, 0:1]
    by1c = colsB[:, 1:2]
    bx2c = colsB[:, 2:3]
    by2c = colsB[:, 3:4]
    area_r = jnp.maximum(bx2 - bx1, 0.0) * jnp.maximum(by2 - by1, 0.0)
    area_c = jnp.maximum(bx2c - bx1c, 0.0) * jnp.maximum(by2c - by1c, 0.0)
    ix1 = jnp.maximum(bx1c, bx1)
    iy1 = jnp.maximum(by1c, by1)
    ix2 = jnp.minimum(bx2c, bx2)
    iy2 = jnp.minimum(by2c, by2)
    inter = jnp.maximum(ix2 - ix1, 0.0) * jnp.maximum(iy2 - iy1, 0.0)
    iou = inter / (area_c + area_r - inter + 1e-9)
    supp = jnp.max(jnp.where(above, iou, 0.0), axis=0, keepdims=True)

    keep = (supp <= _NMS_THRESH) & (score_r > 0.0)
    keepf = keep.astype(jnp.float32)

    pay8 = jnp.concatenate([
        acc[0][None, :] * keepf, acc[1][None, :] * keepf,
        acc[2][None, :] * keepf, acc[3][None, :] * keepf,
        acc[4][None, :] * keepf,
        jnp.where(keep, lb, -1.0),
        jnp.zeros((2, _S), jnp.float32),
    ], axis=0)
    roh = jnp.where(rank == riota, 1.0, 0.0)
    out_ref[0] = _dotp(pay8, roh)


@jax.jit
def kernel(boxes_offset, image_sizes):
    n, c, hh, ww = boxes_offset.shape
    x = boxes_offset.reshape(n, c, hh * ww)
    obj = x[:, _NC + 4::_STRIDE, :]
    sc = jax.nn.sigmoid(obj)
    sf = jnp.where(sc > 0.5, sc, 0.0)
    sz = image_sizes.astype(jnp.float32).reshape(n, 1, 2)

    out = pl.pallas_call(
        _yolo_kernel,
        grid=(n,),
        in_specs=[
            pl.BlockSpec((1, c, hh * ww), lambda i: (i, 0, 0)),
            pl.BlockSpec((1, _A, hh * ww), lambda i: (i, 0, 0)),
            pl.BlockSpec((1, 1, 2), lambda i: (i, 0, 0),
                         memory_space=pltpu.SMEM),
        ],
        out_specs=pl.BlockSpec((1, 8, _S), lambda i: (i, 0, 0)),
        out_shape=jax.ShapeDtypeStruct((n, 8, _S), jnp.float32),
        compiler_params=pltpu.CompilerParams(
            dimension_semantics=("arbitrary",)),
    )(x, sf, sz)

    boxes = jnp.transpose(out[:, 0:4, :_K], (0, 2, 1))
    scores = out[:, 4, :_K]
    labels = out[:, 5, :_K].astype(jnp.int32)
    return boxes, scores, labels
